# Initial kernel scaffold; baseline (speedup 1.0000x reference)
import jax
import jax.numpy as jnp
from jax.experimental import pallas as pl


def kernel(x, edge_index, ln_g1, ln_b1, sw1, bw1, bb1, bias1, bn_g1, bn_b1, ln_g2, ln_b2, sw2, bw2, bb2, bias2, bn_g2, bn_b2, ln_go, ln_bo, swo, bwo, bbo):
    raise NotImplementedError("write your pallas kernel here")


# speedup vs baseline: 12.9176x; 12.9176x over previous
import functools

import jax
import jax.numpy as jnp
from jax import lax
from jax.experimental import pallas as pl
from jax.experimental.pallas import tpu as pltpu
from jax.experimental.pallas import tpu_sc as plsc

_N = 10000
_E = 320000
_F = 128
_H = 128
_C = 40
_G = 4
_LANES = 128
_NB = _E // _LANES
_NC = 2
_NS = 16
_NW = _NC * _NS
_NP = 10240
_RPS = _NP // _NS
_KMAX = (_NB + _NW - 1) // _NW

_GRID_MIN, _GRID_MAX = -2.0, 2.0
_DENOM = (_GRID_MAX - _GRID_MIN) / (_G - 1)
_GRIDS = tuple(_GRID_MIN + i * _DENOM for i in range(_G))


def _sc_degree(col1d, ones_rows, zrows):
    mesh = plsc.VectorSubcoreMesh(core_axis_name="c", subcore_axis_name="s")

    @functools.partial(
        pl.kernel,
        mesh=mesh,
        out_type=jax.ShapeDtypeStruct((_NC, _NP, _H), jnp.float32),
        scratch_types=[
            pltpu.VMEM((_LANES,), jnp.int32),
            pltpu.VMEM((_LANES, _H), jnp.float32),
            pltpu.VMEM_SHARED((_NP, _H), jnp.float32),
        ],
    )
    def run(col_hbm, ones_hbm, z_hbm, out_hbm, colv, onesv, acc):
        c = lax.axis_index("c")
        s = lax.axis_index("s")
        wid = c * _NS + s
        base = pl.multiple_of(s * _RPS, _RPS)
        pltpu.sync_copy(z_hbm, acc.at[pl.ds(base, _RPS)])
        pltpu.sync_copy(ones_hbm, onesv)
        plsc.subcore_barrier()

        def body(k, carry):
            b = k * _NW + wid

            @pl.when(b < _NB)
            def _():
                eb = pl.multiple_of(b * _LANES, _LANES)
                pltpu.sync_copy(col_hbm.at[pl.ds(eb, _LANES)], colv)
                pltpu.sync_copy(onesv, acc.at[colv], add=True)

            return carry

        lax.fori_loop(0, _KMAX, body, 0)
        plsc.subcore_barrier()
        pltpu.sync_copy(acc.at[pl.ds(base, _RPS)],
                        out_hbm.at[c, pl.ds(base, _RPS)])

    return run(col1d, ones_rows, zrows)


def _sc_segment_sum(hs, row1d, col1d, zrows):
    mesh = plsc.VectorSubcoreMesh(core_axis_name="c", subcore_axis_name="s")

    @functools.partial(
        pl.kernel,
        mesh=mesh,
        out_type=jax.ShapeDtypeStruct((_NC, _NP, _H), jnp.float32),
        scratch_types=[
            pltpu.VMEM((_LANES,), jnp.int32),
            pltpu.VMEM((_LANES,), jnp.int32),
            pltpu.VMEM((_LANES, _H), jnp.float32),
            pltpu.VMEM_SHARED((_NP, _H), jnp.float32),
            pltpu.SemaphoreType.DMA,
        ],
    )
    def run(hs_hbm, row_hbm, col_hbm, z_hbm, out_hbm, rowv, colv, rows, acc,
            sem):
        c = lax.axis_index("c")
        s = lax.axis_index("s")
        wid = c * _NS + s
        base = pl.multiple_of(s * _RPS, _RPS)
        pltpu.sync_copy(z_hbm, acc.at[pl.ds(base, _RPS)])
        plsc.subcore_barrier()

        def body(k, carry):
            b = k * _NW + wid

            @pl.when(b < _NB)
            def _():
                eb = pl.multiple_of(b * _LANES, _LANES)
                pltpu.sync_copy(row_hbm.at[pl.ds(eb, _LANES)], rowv)
                pltpu.sync_copy(col_hbm.at[pl.ds(eb, _LANES)], colv)
                pltpu.async_copy(hs_hbm.at[rowv], rows, sem).wait()
                pltpu.sync_copy(rows, acc.at[colv], add=True)

            return carry

        lax.fori_loop(0, _KMAX, body, 0)
        plsc.subcore_barrier()
        pltpu.sync_copy(acc.at[pl.ds(base, _RPS)],
                        out_hbm.at[c, pl.ds(base, _RPS)])

    return run(hs, row1d, col1d, zrows)


def _dis_from_parts(dp):
    deg = dp[0, :, 0:1] + dp[1, :, 0:1] + 1.0
    return lax.rsqrt(deg)


def _fastkan(x, ln_g, ln_b, swr, bwT, bb):
    m = jnp.mean(x, axis=1, keepdims=True)
    v = jnp.mean((x - m) ** 2, axis=1, keepdims=True)
    xn = (x - m) * lax.rsqrt(v + 1e-5) * ln_g + ln_b
    acc = jnp.dot(jax.nn.silu(x), bwT,
                  preferred_element_type=jnp.float32) + bb
    for g0 in range(_G):
        basis = jnp.exp(-(((xn - _GRIDS[g0]) / _DENOM) ** 2))
        acc = acc + jnp.dot(basis, swr[g0],
                            preferred_element_type=jnp.float32)
    return acc


_BLK = 2000
_NSTEP = _N // _BLK


def _full(shape):
    r = len(shape)
    return pl.BlockSpec(shape, lambda i, _r=r: (0,) * _r)


def _rows(shape):
    r = len(shape)
    return pl.BlockSpec((_BLK,) + tuple(shape[1:]),
                        lambda i, _r=r: (i,) + (0,) * (_r - 1))


_DP_SPEC = pl.BlockSpec((2, _BLK, _H), lambda i: (0, i, 0))


def _tc_stage1(x, dp, ln_g, ln_b, swr, bwT, bb):
    def body(x_ref, dp_ref, g_ref, b_ref, swr_ref, bwT_ref, bb_ref, hs_ref):
        dis = _dis_from_parts(dp_ref[...])
        h = _fastkan(x_ref[...], g_ref[...], b_ref[...], swr_ref[...],
                     bwT_ref[...], bb_ref[...])
        hs_ref[...] = h * dis

    return pl.pallas_call(
        body,
        grid=(_NSTEP,),
        in_specs=[_rows(x.shape), _DP_SPEC, _full(ln_g.shape),
                  _full(ln_b.shape), _full(swr.shape), _full(bwT.shape),
                  _full(bb.shape)],
        out_specs=_rows((_N, _H)),
        out_shape=jax.ShapeDtypeStruct((_N, _H), jnp.float32),
    )(x, dp, ln_g, ln_b, swr, bwT, bb)


def _tc_aggstats(p, hs, dp, bias):
    def body(p_ref, hs_ref, dp_ref, bias_ref, agg_ref, st_ref):
        i = pl.program_id(0)
        dis = _dis_from_parts(dp_ref[...])
        agg = (p_ref[0] + p_ref[1] + hs_ref[...]) * dis + bias_ref[...]
        agg_ref[...] = agg

        @pl.when(i == 0)
        def _():
            st_ref[...] = jnp.zeros((2, _H), jnp.float32)

        st_ref[...] += jnp.stack(
            [jnp.sum(agg, axis=0), jnp.sum(agg * agg, axis=0)])

    return pl.pallas_call(
        body,
        grid=(_NSTEP,),
        in_specs=[pl.BlockSpec((2, _BLK, _H), lambda i: (0, i, 0)),
                  _rows((_N, _H)), _DP_SPEC, _full(bias.shape)],
        out_specs=[_rows((_N, _H)), _full((2, _H))],
        out_shape=[jax.ShapeDtypeStruct((_N, _H), jnp.float32),
                   jax.ShapeDtypeStruct((2, _H), jnp.float32)],
    )(p, hs, dp, bias)


def _bn_from_stats(x, st, g, b):
    m = st[0:1] / float(_N)
    v = st[1:2] / float(_N) - m * m
    return (x - m) * lax.rsqrt(v + 1e-5) * g + b


def _tc_stage2(agg, st, dp, bn_g, bn_b, ln_g, ln_b, swr, bwT, bb):
    def body(agg_ref, st_ref, dp_ref, bng_ref, bnb_ref, lng_ref, lnb_ref,
             swr_ref, bwT_ref, bb_ref, h1p_ref, hs2_ref):
        dis = _dis_from_parts(dp_ref[...])
        h1p = _bn_from_stats(agg_ref[...], st_ref[...], bng_ref[...],
                             bnb_ref[...])
        h1p_ref[...] = h1p
        h2 = _fastkan(h1p, lng_ref[...], lnb_ref[...], swr_ref[...],
                      bwT_ref[...], bb_ref[...])
        hs2_ref[...] = h2 * dis

    return pl.pallas_call(
        body,
        grid=(_NSTEP,),
        in_specs=[_rows((_N, _H)), _full((2, _H)), _DP_SPEC,
                  _full(bn_g.shape), _full(bn_b.shape), _full(ln_g.shape),
                  _full(ln_b.shape), _full(swr.shape), _full(bwT.shape),
                  _full(bb.shape)],
        out_specs=[_rows((_N, _H)), _rows((_N, _H))],
        out_shape=[jax.ShapeDtypeStruct((_N, _H), jnp.float32),
                   jax.ShapeDtypeStruct((_N, _H), jnp.float32)],
    )(agg, st, dp, bn_g, bn_b, ln_g, ln_b, swr, bwT, bb)


def _tc_stage3(x, h1p, agg2, st2, bn_g, bn_b, lng_r, lnb_r, swro, bwTo, bbo):
    def body(x_ref, h1p_ref, agg_ref, st_ref, bng_ref, bnb_ref, lng_ref,
             lnb_ref, swr_ref, bwT_ref, bb_ref, out_ref):
        h2p = _bn_from_stats(agg_ref[...], st_ref[...], bng_ref[...],
                             bnb_ref[...])
        pieces = (x_ref[...], h1p_ref[...], h2p)
        din = float(3 * _H)
        m = (sum(jnp.sum(p, axis=1, keepdims=True) for p in pieces)) / din
        ssd = sum(jnp.sum((p - m) ** 2, axis=1, keepdims=True)
                  for p in pieces)
        inv = lax.rsqrt(ssd / din + 1e-5)
        acc = jnp.zeros((_BLK, _C), jnp.float32) + bb_ref[...]
        for pi, piece in enumerate(pieces):
            xn = (piece - m) * inv * lng_ref[pi] + lnb_ref[pi]
            acc = acc + jnp.dot(jax.nn.silu(piece), bwT_ref[pi],
                                preferred_element_type=jnp.float32)
            for g0 in range(_G):
                basis = jnp.exp(-(((xn - _GRIDS[g0]) / _DENOM) ** 2))
                acc = acc + jnp.dot(basis, swr_ref[pi, g0],
                                    preferred_element_type=jnp.float32)
        out_ref[...] = acc

    return pl.pallas_call(
        body,
        grid=(_NSTEP,),
        in_specs=[_rows((_N, _F)), _rows((_N, _H)), _rows((_N, _H)),
                  _full((2, _H)), _full(bn_g.shape), _full(bn_b.shape),
                  _full(lng_r.shape), _full(lnb_r.shape), _full(swro.shape),
                  _full(bwTo.shape), _full(bbo.shape)],
        out_specs=_rows((_N, _C)),
        out_shape=jax.ShapeDtypeStruct((_N, _C), jnp.float32),
    )(x, h1p, agg2, st2, bn_g, bn_b, lng_r, lnb_r, swro, bwTo, bbo)


def kernel(x, edge_index, ln_g1, ln_b1, sw1, bw1, bb1, bias1, bn_g1, bn_b1,
           ln_g2, ln_b2, sw2, bw2, bb2, bias2, bn_g2, bn_b2, ln_go, ln_bo,
           swo, bwo, bbo):
    row1d = edge_index[0]
    col1d = edge_index[1]
    zrows = jnp.zeros((_RPS, _H), jnp.float32)
    ones_rows = jnp.ones((_LANES, _H), jnp.float32)

    swr1 = jnp.transpose(sw1.reshape(_H, _F, _G), (2, 1, 0))
    swr2 = jnp.transpose(sw2.reshape(_H, _H, _G), (2, 1, 0))
    swro = jnp.transpose(swo.reshape(_C, 3, _H, _G), (1, 3, 2, 0))
    bwTo = jnp.transpose(bwo.reshape(_C, 3, _H), (1, 2, 0))

    dp = _sc_degree(col1d, ones_rows, zrows)[:, :_N]
    hs1 = _tc_stage1(x, dp, ln_g1, ln_b1, swr1, bw1.T, bb1)
    p1 = _sc_segment_sum(hs1, row1d, col1d, zrows)[:, :_N]
    agg1, st1 = _tc_aggstats(p1, hs1, dp, bias1)
    h1p, hs2 = _tc_stage2(agg1, st1, dp, bn_g1, bn_b1, ln_g2, ln_b2,
                          swr2, bw2.T, bb2)
    p2 = _sc_segment_sum(hs2, row1d, col1d, zrows)[:, :_N]
    agg2, st2 = _tc_aggstats(p2, hs2, dp, bias2)
    return _tc_stage3(x, h1p, agg2, st2, bn_g2, bn_b2,
                      ln_go.reshape(3, _H), ln_bo.reshape(3, _H), swro,
                      bwTo, bbo)
